# baseline (speedup 1.0000x reference)
import functools

import jax
import jax.numpy as jnp
from jax import lax
from jax.experimental import pallas as pl
from jax.experimental.pallas import tpu as pltpu
from jax.experimental.pallas import tpu_sc as plsc

F32 = jnp.float32
I32 = jnp.int32
HI = lax.Precision.HIGHEST

_B, _S = 4, 256
_N1, _N2, _PD, _OUT = 64, 64, 32, 64
_D, _A = 8, 6
_NE = _D * _A
_NROW = _D * 64
_SENT = _NROW
_TJ = 16

_L = 16
_CB = 544
_E = _S * _D


def _codes_body(in1_ref, in2_ref, pe_ref, anch_ref, out_ref, d1d_ref, dp_ref):
    jb = pl.program_id(1)
    a0 = anch_ref[0:1, :]
    a1 = anch_ref[1:2, :]

    def sel(base, n, neg=False):
        io = lax.broadcasted_iota(I32, (n, _NE), 0) + base
        p, q = (a1, a0) if neg else (a0, a1)
        return (io == p).astype(F32) - (io == q).astype(F32)

    @pl.when(jb == 0)
    def _tables():
        ed1 = sel(0, _N1)
        edp = sel(_N1 + _N2, _PD)
        d1 = jnp.dot(in1_ref[0], ed1, precision=HI)
        ri = lax.broadcasted_iota(I32, (_S, _S), 0)
        ci = lax.broadcasted_iota(I32, (_S, _S), 1)
        jmat = (ri + ci == _S - 1).astype(F32)
        d1r = jnp.dot(jmat, d1, precision=HI)
        d1r_up = jnp.concatenate([d1r[1:], d1r[:1]], axis=0)
        pair0 = jnp.concatenate([d1r, d1r_up], axis=1)
        d1d_ref[0:_S, :] = pair0
        d1d_ref[_S:2 * _S, :] = pair0
        dp = jnp.dot(pe_ref[...], edp, precision=HI)
        dp_ref[...] = jnp.concatenate([dp, dp], axis=1)

    ed2n = sel(_N1, _N2, neg=True)
    d2n = jnp.dot(in2_ref[0], ed2n, precision=HI)
    dp96 = dp_ref[...]

    BF = jnp.bfloat16
    m_io = lax.broadcasted_iota(I32, (2 * _D, 2 * _NE), 0)
    e_io = lax.broadcasted_iota(I32, (2 * _D, 2 * _NE), 1)
    w96t = jnp.where(e_io // _A == m_io, (1 << (e_io % _A)), 0).astype(BF)

    _HC = _S // 2
    row_io = lax.broadcasted_iota(I32, (2 * _D, _HC), 0)
    r_io = lax.broadcasted_iota(I32, (2 * _D, _HC), 1)
    lane64 = (row_io % _D) * 64
    rpar = (r_io % 2) * _CB

    def pair_chunk(jj, roff):
        j0 = jb * _TJ + jj
        j1 = j0 + 1
        sh = d1d_ref[pl.ds(_S - 2 - j0 + roff, _HC), :]
        d2pair = jnp.concatenate([d2n[jj + 1:jj + 2], d2n[jj:jj + 1]],
                                 axis=1)
        bits = ((sh + dp96[roff:roff + _HC]) > d2pair).astype(BF)
        codes_f = lax.dot_general(w96t, bits, (((1,), (1,)), ((), ())),
                                  preferred_element_type=F32)
        codes = codes_f.astype(I32) + lane64
        jlim = jnp.where(row_io < _D, j1, j0)
        rr = r_io + roff
        codes = jnp.where((rr >= 1) & (rr <= jlim), codes, _SENT) + rpar
        out_ref[jj, :, roff:roff + _HC] = codes[_D:, :]
        out_ref[jj + 1, :, roff:roff + _HC] = codes[:_D, :]

    for jj in range(0, _TJ, 2):
        pair_chunk(jj, 0)

    @pl.when(jb >= _S // (2 * _TJ))
    def _upper():
        for jj in range(0, _TJ, 2):
            pair_chunk(jj, _HC)


def _run_codes(input_1, input_2, pe_pad, anch2):
    grid = (_B, _S // _TJ)
    return pl.pallas_call(
        _codes_body,
        grid=grid,
        in_specs=[
            pl.BlockSpec((1, _S, _N1), lambda b, jb: (b, 0, 0)),
            pl.BlockSpec((1, _TJ, _N2), lambda b, jb: (b, jb, 0)),
            pl.BlockSpec((_S, _PD), lambda b, jb: (0, 0)),
            pl.BlockSpec((8, _NE), lambda b, jb: (0, 0)),
        ],
        out_specs=pl.BlockSpec((_TJ, _D, _S),
                               lambda b, jb: (b * (_S // _TJ) + jb, 0, 0)),
        out_shape=jax.ShapeDtypeStruct((_B * _S, _D, _S), I32),
        scratch_shapes=[pltpu.VMEM((2 * _S, 2 * _NE), F32),
                        pltpu.VMEM((_S, 2 * _NE), F32)],
    )(input_1, input_2, pe_pad, anch2)


def _run_hist(codes2d):
    info = plsc.get_sparse_core_info()
    nw = info.num_cores * info.num_subcores
    rows = _B * _S
    rpw = rows // nw

    mesh = plsc.VectorSubcoreMesh(core_axis_name="c", subcore_axis_name="s")

    @functools.partial(
        pl.kernel,
        mesh=mesh,
        out_type=jax.ShapeDtypeStruct((rows, _NROW), F32),
        scratch_types=[
            pltpu.VMEM((2 * _D, _S), I32),
            pltpu.VMEM((2 * _CB,), F32),
            pltpu.VMEM((2, _NROW), F32),
            pltpu.SemaphoreType.DMA,
            pltpu.SemaphoreType.DMA,
        ],
        compiler_params=pltpu.CompilerParams(needs_layout_passes=False),
    )
    def hist(codes_hbm, out_hbm, idx_v, cnt_v, o_v, sem, sem_out):
        wid = lax.axis_index("s") * info.num_cores + lax.axis_index("c")
        ones = jnp.full((_L,), 1.0, F32)
        lane = lax.iota(I32, _L)
        d_vec = lax.rem(lane, _D)
        r_vec = lane // _D

        pltpu.async_copy(codes_hbm.at[wid], idx_v.at[pl.ds(0, _D)], sem)

        def row_body(rr, carry):
            p8 = lax.rem(rr, 2) * _D
            q = lax.rem(rr, 2)
            row = rr * nw + wid
            pltpu.make_async_copy(codes_hbm.at[row], idx_v.at[pl.ds(p8, _D)],
                                  sem).wait()

            @pl.when(rr + 1 < rpw)
            def _prefetch():
                pltpu.async_copy(codes_hbm.at[row + nw],
                                 idx_v.at[pl.ds(_D - p8, _D)], sem)

            for t in range(2 * _CB // _L):
                cnt_v[pl.ds(t * _L, _L)] = jnp.zeros((_L,), F32)

            j = lax.rem(row, _S)
            ntrip = (j + 1 + 7) // 8
            dp8 = d_vec + p8
            eight = jnp.full((_L,), 8, I32)

            def scat(k, carry):
                rs = carry
                out = []
                for u in range(4):
                    iv = plsc.load_gather(idx_v, [dp8, rs[u]])
                    plsc.addupdate_scatter(cnt_v, [iv], ones)
                    out.append(rs[u] + eight)
                return tuple(out)

            lax.fori_loop(0, ntrip, scat,
                          tuple(r_vec + 2 * u for u in range(4)))

            @pl.when(rr >= 2)
            def _drain():
                pltpu.make_async_copy(o_v.at[q], out_hbm.at[row - 2 * nw],
                                      sem_out).wait()

            for t in range(_NROW // _L):
                o_v[q, pl.ds(t * _L, _L)] = (cnt_v[pl.ds(t * _L, _L)]
                                             + cnt_v[pl.ds(_CB + t * _L, _L)])

            pltpu.async_copy(o_v.at[q], out_hbm.at[row], sem_out)
            return carry

        lax.fori_loop(0, rpw, row_body, 0)
        for rr in (rpw - 2, rpw - 1):
            pltpu.make_async_copy(o_v.at[rr % 2], out_hbm.at[rr * nw + wid],
                                  sem_out).wait()

    return hist(codes2d)


def _mm_body(cnt_ref, lut_ref, out_ref):
    out_ref[0] = jnp.dot(cnt_ref[...], lut_ref[...], precision=HI)


def _run_mm(counts, lut_flat):
    return pl.pallas_call(
        _mm_body,
        grid=(_B,),
        in_specs=[
            pl.BlockSpec((_S, _NROW), lambda g: (g, 0)),
            pl.BlockSpec((_NROW, _OUT), lambda g: (0, 0)),
        ],
        out_specs=pl.BlockSpec((1, _S, _OUT), lambda g: (g, 0, 0)),
        out_shape=jax.ShapeDtypeStruct((_B, _S, _OUT), F32),
    )(counts, lut_flat)


def kernel(input_1, input_2, pos_emb, anchors, lut_w):
    bb, ss, _ = input_1.shape
    pe_pad = jnp.concatenate([pos_emb, pos_emb[-1:]], axis=0)
    a0 = anchors[..., 0].reshape(-1).astype(I32)
    a1 = anchors[..., 1].reshape(-1).astype(I32)
    anch2 = jnp.zeros((8, _NE), I32).at[0].set(a0).at[1].set(a1)

    codes = _run_codes(input_1, input_2, pe_pad, anch2)
    counts = _run_hist(codes)
    return _run_mm(counts, lut_w.reshape(_NROW, _OUT))


# speedup vs baseline: 1.0369x; 1.0369x over previous
import functools

import jax
import jax.numpy as jnp
from jax import lax
from jax.experimental import pallas as pl
from jax.experimental.pallas import tpu as pltpu
from jax.experimental.pallas import tpu_sc as plsc

F32 = jnp.float32
I32 = jnp.int32
HI = lax.Precision.HIGHEST

_B, _S = 4, 256
_N1, _N2, _PD, _OUT = 64, 64, 32, 64
_D, _A = 8, 6
_NE = _D * _A
_NROW = _D * 64
_SENT = _NROW
_TJ = 16

_L = 16
_CB = 544
_E = _S * _D


def _codes_body(in1_ref, in2_ref, pe_ref, anch_ref, out_ref, d1d_ref, dp_ref):
    jb = pl.program_id(1)
    a0 = anch_ref[0:1, :]
    a1 = anch_ref[1:2, :]

    def sel(base, n, neg=False):
        io = lax.broadcasted_iota(I32, (n, _NE), 0) + base
        p, q = (a1, a0) if neg else (a0, a1)
        return (io == p).astype(F32) - (io == q).astype(F32)

    @pl.when(jb == 0)
    def _tables():
        ed1 = sel(0, _N1)
        edp = sel(_N1 + _N2, _PD)
        d1 = jnp.dot(in1_ref[0], ed1, precision=HI)
        ri = lax.broadcasted_iota(I32, (_S, _S), 0)
        ci = lax.broadcasted_iota(I32, (_S, _S), 1)
        jmat = (ri + ci == _S - 1).astype(F32)
        d1r = jnp.dot(jmat, d1, precision=HI)
        d1r_up = jnp.concatenate([d1r[1:], d1r[:1]], axis=0)
        pair0 = jnp.concatenate([d1r, d1r_up], axis=1)
        d1d_ref[0:_S, :] = pair0
        d1d_ref[_S:2 * _S, :] = pair0
        dp = jnp.dot(pe_ref[...], edp, precision=HI)
        dp_ref[...] = jnp.concatenate([dp, dp], axis=1)

    ed2n = sel(_N1, _N2, neg=True)
    d2n = jnp.dot(in2_ref[0], ed2n, precision=HI)
    dp96 = dp_ref[...]

    BF = jnp.bfloat16
    m_io = lax.broadcasted_iota(I32, (2 * _D, 2 * _NE), 0)
    e_io = lax.broadcasted_iota(I32, (2 * _D, 2 * _NE), 1)
    w96t = jnp.where(e_io // _A == m_io, (1 << (e_io % _A)), 0).astype(BF)

    row_io = lax.broadcasted_iota(I32, (2 * _D, _S), 0)
    r_io = lax.broadcasted_iota(I32, (2 * _D, _S), 1)
    lane64 = (row_io % _D) * 64
    rpar = (r_io % 2) * _CB

    for jj in range(0, _TJ, 2):
        j0 = jb * _TJ + jj
        j1 = j0 + 1
        sh = d1d_ref[pl.ds(_S - 2 - j0, _S), :]
        d2pair = jnp.concatenate([d2n[jj + 1:jj + 2], d2n[jj:jj + 1]],
                                 axis=1)
        bits = ((sh + dp96) > d2pair).astype(BF)
        codes_f = lax.dot_general(w96t, bits, (((1,), (1,)), ((), ())),
                                  preferred_element_type=F32)
        codes = codes_f.astype(I32) + lane64
        jlim = jnp.where(row_io < _D, j1, j0)
        codes = jnp.where((r_io >= 1) & (r_io <= jlim), codes, _SENT) + rpar
        out_ref[jj] = codes[_D:, :]
        out_ref[jj + 1] = codes[:_D, :]


def _run_codes(input_1, input_2, pe_pad, anch2):
    grid = (_B, _S // _TJ)
    return pl.pallas_call(
        _codes_body,
        grid=grid,
        in_specs=[
            pl.BlockSpec((1, _S, _N1), lambda b, jb: (b, 0, 0)),
            pl.BlockSpec((1, _TJ, _N2), lambda b, jb: (b, jb, 0)),
            pl.BlockSpec((_S, _PD), lambda b, jb: (0, 0)),
            pl.BlockSpec((8, _NE), lambda b, jb: (0, 0)),
        ],
        out_specs=pl.BlockSpec((_TJ, _D, _S),
                               lambda b, jb: (b * (_S // _TJ) + jb, 0, 0)),
        out_shape=jax.ShapeDtypeStruct((_B * _S, _D, _S), I32),
        scratch_shapes=[pltpu.VMEM((2 * _S, 2 * _NE), F32),
                        pltpu.VMEM((_S, 2 * _NE), F32)],
    )(input_1, input_2, pe_pad, anch2)


def _run_hist(codes2d):
    info = plsc.get_sparse_core_info()
    nw = info.num_cores * info.num_subcores
    rows = _B * _S
    rpw = rows // nw

    mesh = plsc.VectorSubcoreMesh(core_axis_name="c", subcore_axis_name="s")

    @functools.partial(
        pl.kernel,
        mesh=mesh,
        out_type=jax.ShapeDtypeStruct((rows, _NROW), F32),
        scratch_types=[
            pltpu.VMEM((2 * _D, _S), I32),
            pltpu.VMEM((2 * _CB,), F32),
            pltpu.VMEM((2, _NROW), F32),
            pltpu.SemaphoreType.DMA,
            pltpu.SemaphoreType.DMA,
        ],
        compiler_params=pltpu.CompilerParams(needs_layout_passes=False),
    )
    def hist(codes_hbm, out_hbm, idx_v, cnt_v, o_v, sem, sem_out):
        wid = lax.axis_index("s") * info.num_cores + lax.axis_index("c")
        ones = jnp.full((_L,), 1.0, F32)
        lane = lax.iota(I32, _L)
        d_vec = lax.rem(lane, _D)
        r_vec = lane // _D

        pltpu.async_copy(codes_hbm.at[wid], idx_v.at[pl.ds(0, _D)], sem)

        def row_body(rr, carry):
            p8 = lax.rem(rr, 2) * _D
            q = lax.rem(rr, 2)
            row = rr * nw + wid
            pltpu.make_async_copy(codes_hbm.at[row], idx_v.at[pl.ds(p8, _D)],
                                  sem).wait()

            @pl.when(rr + 1 < rpw)
            def _prefetch():
                pltpu.async_copy(codes_hbm.at[row + nw],
                                 idx_v.at[pl.ds(_D - p8, _D)], sem)

            for t in range(2 * _CB // _L):
                cnt_v[pl.ds(t * _L, _L)] = jnp.zeros((_L,), F32)

            j = lax.rem(row, _S)
            ntrip = (j + 1 + 7) // 8
            dp8 = d_vec + p8
            eight = jnp.full((_L,), 8, I32)

            def scat(k, carry):
                rs = carry
                out = []
                for u in range(4):
                    iv = plsc.load_gather(idx_v, [dp8, rs[u]])
                    plsc.addupdate_scatter(cnt_v, [iv], ones)
                    out.append(rs[u] + eight)
                return tuple(out)

            lax.fori_loop(0, ntrip, scat,
                          tuple(r_vec + 2 * u for u in range(4)))

            @pl.when(rr >= 2)
            def _drain():
                pltpu.make_async_copy(o_v.at[q], out_hbm.at[row - 2 * nw],
                                      sem_out).wait()

            for t in range(_NROW // _L):
                o_v[q, pl.ds(t * _L, _L)] = (cnt_v[pl.ds(t * _L, _L)]
                                             + cnt_v[pl.ds(_CB + t * _L, _L)])

            pltpu.async_copy(o_v.at[q], out_hbm.at[row], sem_out)
            return carry

        lax.fori_loop(0, rpw, row_body, 0)
        for rr in (rpw - 2, rpw - 1):
            pltpu.make_async_copy(o_v.at[rr % 2], out_hbm.at[rr * nw + wid],
                                  sem_out).wait()

    return hist(codes2d)


def _mm_body(cnt_ref, lut_ref, out_ref):
    out_ref[0] = jnp.dot(cnt_ref[...], lut_ref[...], precision=HI)


def _run_mm(counts, lut_flat):
    return pl.pallas_call(
        _mm_body,
        grid=(_B,),
        in_specs=[
            pl.BlockSpec((_S, _NROW), lambda g: (g, 0)),
            pl.BlockSpec((_NROW, _OUT), lambda g: (0, 0)),
        ],
        out_specs=pl.BlockSpec((1, _S, _OUT), lambda g: (g, 0, 0)),
        out_shape=jax.ShapeDtypeStruct((_B, _S, _OUT), F32),
    )(counts, lut_flat)


def kernel(input_1, input_2, pos_emb, anchors, lut_w):
    bb, ss, _ = input_1.shape
    pe_pad = jnp.concatenate([pos_emb, pos_emb[-1:]], axis=0)
    a0 = anchors[..., 0].reshape(-1).astype(I32)
    a1 = anchors[..., 1].reshape(-1).astype(I32)
    anch2 = jnp.zeros((8, _NE), I32).at[0].set(a0).at[1].set(a1)

    codes = _run_codes(input_1, input_2, pe_pad, anch2)
    counts = _run_hist(codes)
    return _run_mm(counts, lut_w.reshape(_NROW, _OUT))


# speedup vs baseline: 1.0489x; 1.0115x over previous
import functools

import jax
import jax.numpy as jnp
from jax import lax
from jax.experimental import pallas as pl
from jax.experimental.pallas import tpu as pltpu
from jax.experimental.pallas import tpu_sc as plsc

F32 = jnp.float32
I32 = jnp.int32
HI = lax.Precision.HIGHEST

_B, _S = 4, 256
_N1, _N2, _PD, _OUT = 64, 64, 32, 64
_D, _A = 8, 6
_NE = _D * _A
_NROW = _D * 64
_SENT = _NROW
_TJ = 16

_L = 16
_CB = 544
_E = _S * _D


def _codes_body(in1_ref, in2_ref, pe_ref, anch_ref, out_ref, d1d_ref, dp_ref):
    jb = pl.program_id(1)
    a0 = anch_ref[0:1, :]
    a1 = anch_ref[1:2, :]

    def sel(base, n, neg=False):
        io = lax.broadcasted_iota(I32, (n, _NE), 0) + base
        p, q = (a1, a0) if neg else (a0, a1)
        return (io == p).astype(F32) - (io == q).astype(F32)

    @pl.when(jb == 0)
    def _tables():
        ed1 = sel(0, _N1)
        edp = sel(_N1 + _N2, _PD)
        d1 = jnp.dot(in1_ref[0], ed1, precision=HI)
        ri = lax.broadcasted_iota(I32, (_S, _S), 0)
        ci = lax.broadcasted_iota(I32, (_S, _S), 1)
        jmat = (ri + ci == _S - 1).astype(F32)
        d1r = jnp.dot(jmat, d1, precision=HI)
        d1r_up = jnp.concatenate([d1r[1:], d1r[:1]], axis=0)
        pair0 = jnp.concatenate([d1r, d1r_up], axis=1)
        d1d_ref[0:_S, :] = pair0
        d1d_ref[_S:2 * _S, :] = pair0
        dp = jnp.dot(pe_ref[...], edp, precision=HI)
        dp_ref[...] = jnp.concatenate([dp, dp], axis=1)

    ed2n = sel(_N1, _N2, neg=True)
    d2n = jnp.dot(in2_ref[0], ed2n, precision=HI)
    dp96 = dp_ref[...]

    BF = jnp.bfloat16
    m_io = lax.broadcasted_iota(I32, (2 * _D, 2 * _NE), 0)
    e_io = lax.broadcasted_iota(I32, (2 * _D, 2 * _NE), 1)
    w96t = jnp.where(e_io // _A == m_io, (1 << (e_io % _A)), 0).astype(BF)

    row_io = lax.broadcasted_iota(I32, (2 * _D, _S), 0)
    r_io = lax.broadcasted_iota(I32, (2 * _D, _S), 1)
    lane64 = (row_io % _D) * 64
    rpar = (r_io % 2) * _CB

    for jj in range(0, _TJ, 2):
        j0 = jb * _TJ + jj
        j1 = j0 + 1
        sh = d1d_ref[pl.ds(_S - 2 - j0, _S), :]
        d2pair = jnp.concatenate([d2n[jj + 1:jj + 2], d2n[jj:jj + 1]],
                                 axis=1)
        bits = ((sh + dp96) > d2pair).astype(BF)
        codes_f = lax.dot_general(w96t, bits, (((1,), (1,)), ((), ())),
                                  preferred_element_type=F32)
        codes = codes_f.astype(I32) + lane64
        jlim = jnp.where(row_io < _D, j1, j0)
        codes = jnp.where((r_io >= 1) & (r_io <= jlim), codes, _SENT) + rpar
        out_ref[jj] = codes[_D:, :]
        out_ref[jj + 1] = codes[:_D, :]


def _run_codes(input_1, input_2, pe_pad, anch2):
    grid = (_B, _S // _TJ)
    return pl.pallas_call(
        _codes_body,
        grid=grid,
        in_specs=[
            pl.BlockSpec((1, _S, _N1), lambda b, jb: (b, 0, 0)),
            pl.BlockSpec((1, _TJ, _N2), lambda b, jb: (b, jb, 0)),
            pl.BlockSpec((_S, _PD), lambda b, jb: (0, 0)),
            pl.BlockSpec((8, _NE), lambda b, jb: (0, 0)),
        ],
        out_specs=pl.BlockSpec((_TJ, _D, _S),
                               lambda b, jb: (b * (_S // _TJ) + jb, 0, 0)),
        out_shape=jax.ShapeDtypeStruct((_B * _S, _D, _S), I32),
        scratch_shapes=[pltpu.VMEM((2 * _S, 2 * _NE), F32),
                        pltpu.VMEM((_S, 2 * _NE), F32)],
    )(input_1, input_2, pe_pad, anch2)


def _run_hist(codes2d):
    info = plsc.get_sparse_core_info()
    nw = info.num_cores * info.num_subcores
    rows = _B * _S
    rpw = rows // nw

    mesh = plsc.VectorSubcoreMesh(core_axis_name="c", subcore_axis_name="s")

    @functools.partial(
        pl.kernel,
        mesh=mesh,
        out_type=jax.ShapeDtypeStruct((rows, _NROW), F32),
        scratch_types=[
            pltpu.VMEM((4 * _D, _S), I32),
            pltpu.VMEM((2 * _CB,), F32),
            pltpu.VMEM((2, _NROW), F32),
            pltpu.SemaphoreType.DMA,
            pltpu.SemaphoreType.DMA,
        ],
        compiler_params=pltpu.CompilerParams(needs_layout_passes=False),
    )
    def hist(codes_hbm, out_hbm, idx_v, cnt_v, o_v, sem, sem_out):
        wid = lax.axis_index("s") * info.num_cores + lax.axis_index("c")
        ones = jnp.full((_L,), 1.0, F32)
        lane = lax.iota(I32, _L)
        d_vec = lax.rem(lane, _D)
        r_vec = lane // _D

        for t in range(3):
            pltpu.async_copy(codes_hbm.at[t * nw + wid],
                             idx_v.at[pl.ds(t * _D, _D)], sem)

        def row_body(rr, carry):
            p8 = lax.rem(rr, 4) * _D
            q = lax.rem(rr, 2)
            row = rr * nw + wid
            pltpu.make_async_copy(codes_hbm.at[row], idx_v.at[pl.ds(p8, _D)],
                                  sem).wait()

            @pl.when(rr + 3 < rpw)
            def _prefetch():
                pltpu.async_copy(codes_hbm.at[row + 3 * nw],
                                 idx_v.at[pl.ds(lax.rem(rr + 3, 4) * _D, _D)],
                                 sem)

            for t in range(2 * _CB // _L):
                cnt_v[pl.ds(t * _L, _L)] = jnp.zeros((_L,), F32)

            j = lax.rem(row, _S)
            ntrip = (j + 1 + 7) // 8
            dp8 = d_vec + p8
            eight = jnp.full((_L,), 8, I32)

            def scat(k, carry):
                rs = carry
                out = []
                for u in range(4):
                    iv = plsc.load_gather(idx_v, [dp8, rs[u]])
                    plsc.addupdate_scatter(cnt_v, [iv], ones)
                    out.append(rs[u] + eight)
                return tuple(out)

            lax.fori_loop(0, ntrip, scat,
                          tuple(r_vec + 2 * u for u in range(4)))

            @pl.when(rr >= 2)
            def _drain():
                pltpu.make_async_copy(o_v.at[q], out_hbm.at[row - 2 * nw],
                                      sem_out).wait()

            for t in range(_NROW // _L):
                o_v[q, pl.ds(t * _L, _L)] = (cnt_v[pl.ds(t * _L, _L)]
                                             + cnt_v[pl.ds(_CB + t * _L, _L)])

            pltpu.async_copy(o_v.at[q], out_hbm.at[row], sem_out)
            return carry

        lax.fori_loop(0, rpw, row_body, 0)
        for rr in (rpw - 2, rpw - 1):
            pltpu.make_async_copy(o_v.at[rr % 2], out_hbm.at[rr * nw + wid],
                                  sem_out).wait()

    return hist(codes2d)


def _mm_body(cnt_ref, lut_ref, out_ref):
    out_ref[0] = jnp.dot(cnt_ref[...], lut_ref[...], precision=HI)


def _run_mm(counts, lut_flat):
    return pl.pallas_call(
        _mm_body,
        grid=(_B,),
        in_specs=[
            pl.BlockSpec((_S, _NROW), lambda g: (g, 0)),
            pl.BlockSpec((_NROW, _OUT), lambda g: (0, 0)),
        ],
        out_specs=pl.BlockSpec((1, _S, _OUT), lambda g: (g, 0, 0)),
        out_shape=jax.ShapeDtypeStruct((_B, _S, _OUT), F32),
    )(counts, lut_flat)


def kernel(input_1, input_2, pos_emb, anchors, lut_w):
    bb, ss, _ = input_1.shape
    pe_pad = jnp.concatenate([pos_emb, pos_emb[-1:]], axis=0)
    a0 = anchors[..., 0].reshape(-1).astype(I32)
    a1 = anchors[..., 1].reshape(-1).astype(I32)
    anch2 = jnp.zeros((8, _NE), I32).at[0].set(a0).at[1].set(a1)

    codes = _run_codes(input_1, input_2, pe_pad, anch2)
    counts = _run_hist(codes)
    return _run_mm(counts, lut_w.reshape(_NROW, _OUT))
